# initial kernel scaffold (unmeasured)
import jax
import jax.numpy as jnp
from jax import lax
from jax.experimental import pallas as pl
from jax.experimental.pallas import tpu as pltpu

B, S, H, Dh, Dr = 4, 256, 32, 128, 64
D = 4096
DCL = 128
M = B * S
F32 = jnp.float32
SCALE = float((Dh + Dr) ** -0.5)


def _matmul_body(x_ref, w_ref, o_ref):
    o_ref[:, :] = jnp.dot(x_ref[:, :], w_ref[:, :],
                          preferred_element_type=F32)


def _matmul(x, w, block_n):
    m, k = x.shape
    _, n = w.shape
    return pl.pallas_call(
        _matmul_body,
        grid=(n // block_n,),
        in_specs=[
            pl.BlockSpec((m, k), lambda j: (0, 0)),
            pl.BlockSpec((k, block_n), lambda j: (0, j)),
        ],
        out_specs=pl.BlockSpec((m, block_n), lambda j: (0, j)),
        out_shape=jax.ShapeDtypeStruct((m, n), F32),
    )(x, w)


def _exchange_body(c_ref, wuk_ref, wuv_ref, c_out, wuk_out, wuv_out,
                   local_sems, send_sems, recv_sems):
    my_x = lax.axis_index("x")
    my_y = lax.axis_index("y")
    my_z = lax.axis_index("z")
    peer = (my_x, my_y, 1 - my_z)

    barrier = pltpu.get_barrier_semaphore()
    pl.semaphore_signal(barrier, inc=1, device_id=peer,
                        device_id_type=pl.DeviceIdType.MESH)
    pl.semaphore_wait(barrier, 1)

    copies = []
    for i, (src, dst) in enumerate(
        [(c_ref, c_out), (wuk_ref, wuk_out), (wuv_ref, wuv_out)]
    ):
        local = pltpu.make_async_copy(src, dst.at[my_z], local_sems.at[i])
        local.start()
        rdma = pltpu.make_async_remote_copy(
            src_ref=src,
            dst_ref=dst.at[my_z],
            send_sem=send_sems.at[i],
            recv_sem=recv_sems.at[i],
            device_id=peer,
            device_id_type=pl.DeviceIdType.MESH,
        )
        rdma.start()
        copies.append((local, rdma))
    for local, rdma in copies:
        local.wait()
        rdma.wait()


def _exchange(c_half, wuk, wuv):
    return pl.pallas_call(
        _exchange_body,
        out_shape=[
            jax.ShapeDtypeStruct((2, M, DCL), F32),
            jax.ShapeDtypeStruct((2, DCL, D), F32),
            jax.ShapeDtypeStruct((2, DCL, D), F32),
        ],
        in_specs=[pl.BlockSpec(memory_space=pltpu.VMEM)] * 3,
        out_specs=[pl.BlockSpec(memory_space=pltpu.VMEM)] * 3,
        scratch_shapes=[
            pltpu.SemaphoreType.DMA((3,)),
            pltpu.SemaphoreType.DMA((3,)),
            pltpu.SemaphoreType.DMA((3,)),
        ],
        compiler_params=pltpu.CompilerParams(collective_id=0),
    )(c_half, wuk, wuv)


def _kv_body(c_ref, wuk_ref, wuv_ref, k_ref, v_ref):
    c0 = c_ref[0]
    c1 = c_ref[1]
    k_ref[:, :] = (jnp.dot(c0, wuk_ref[0], preferred_element_type=F32)
                   + jnp.dot(c1, wuk_ref[1], preferred_element_type=F32))
    v_ref[:, :] = (jnp.dot(c0, wuv_ref[0], preferred_element_type=F32)
                   + jnp.dot(c1, wuv_ref[1], preferred_element_type=F32))


def _kv(c_st, wuk_st, wuv_st, block_n=512):
    return pl.pallas_call(
        _kv_body,
        grid=(D // block_n,),
        in_specs=[
            pl.BlockSpec((2, M, DCL), lambda j: (0, 0, 0)),
            pl.BlockSpec((2, DCL, block_n), lambda j: (0, 0, j)),
            pl.BlockSpec((2, DCL, block_n), lambda j: (0, 0, j)),
        ],
        out_specs=[
            pl.BlockSpec((M, block_n), lambda j: (0, j)),
            pl.BlockSpec((M, block_n), lambda j: (0, j)),
        ],
        out_shape=[
            jax.ShapeDtypeStruct((M, D), F32),
            jax.ShapeDtypeStruct((M, D), F32),
        ],
    )(c_st, wuk_st, wuv_st)


def _attn_body(q_ref, k_ref, v_ref, qr_ref, kr_ref, o_ref):
    dn = (((1,), (1,)), ((), ()))
    s = lax.dot_general(q_ref[:, :], k_ref[:, :], dn,
                        preferred_element_type=F32)
    s += lax.dot_general(qr_ref[:, :], kr_ref[:, :], dn,
                         preferred_element_type=F32)
    s *= SCALE
    m = jnp.max(s, axis=1, keepdims=True)
    p = jnp.exp(s - m)
    p /= jnp.sum(p, axis=1, keepdims=True)
    o_ref[:, :] = jnp.dot(p, v_ref[:, :], preferred_element_type=F32)


def _attention(q2d, k2d, v2d, qr2d, kr2d):
    return pl.pallas_call(
        _attn_body,
        grid=(B, H),
        in_specs=[
            pl.BlockSpec((S, Dh), lambda b, h: (b, h)),
            pl.BlockSpec((S, Dh), lambda b, h: (b, h)),
            pl.BlockSpec((S, Dh), lambda b, h: (b, h)),
            pl.BlockSpec((S, Dr), lambda b, h: (b, h)),
            pl.BlockSpec((S, Dr), lambda b, h: (b, 0)),
        ],
        out_specs=pl.BlockSpec((S, Dh), lambda b, h: (b, h)),
        out_shape=jax.ShapeDtypeStruct((M, D), F32),
    )(q2d, k2d, v2d, qr2d, kr2d)


def kernel(x, Wdkv, Wuk, Wuv, Wq, Wqr, Wkr, Wo):
    x2d = x.reshape(M, D)
    c_half = _matmul(x2d, Wdkv, DCL)
    c_st, wuk_st, wuv_st = _exchange(c_half, Wuk, Wuv)
    k2d, v2d = _kv(c_st, wuk_st, wuv_st)
    q2d = _matmul(x2d, Wq, 512)
    qr2d = _matmul(x2d, Wqr, 512)
    kr2d = _matmul(x2d, Wkr, 64)
    o2d = _attention(q2d, k2d, v2d, qr2d, kr2d)
    out2d = _matmul(o2d, Wo, 512)
    return out2d.reshape(B, S, D)


# baseline (device time: 298247 ns/iter reference)
import jax
import jax.numpy as jnp
from jax import lax
from jax.experimental import pallas as pl
from jax.experimental.pallas import tpu as pltpu

B, S, H, Dh, Dr = 4, 256, 32, 128, 64
D = 4096
DCL = 128
M = B * S
F32 = jnp.float32
SCALE = float((Dh + Dr) ** -0.5)


def _matmul_body(x_ref, w_ref, o_ref):
    o_ref[:, :] = jnp.dot(x_ref[:, :], w_ref[:, :],
                          preferred_element_type=F32)


def _matmul(x, w, block_n):
    m, k = x.shape
    _, n = w.shape
    return pl.pallas_call(
        _matmul_body,
        grid=(n // block_n,),
        in_specs=[
            pl.BlockSpec((m, k), lambda j: (0, 0)),
            pl.BlockSpec((k, block_n), lambda j: (0, j)),
        ],
        out_specs=pl.BlockSpec((m, block_n), lambda j: (0, j)),
        out_shape=jax.ShapeDtypeStruct((m, n), F32),
        compiler_params=pltpu.CompilerParams(
            vmem_limit_bytes=48 * 1024 * 1024),
    )(x, w)


def _exchange_body(c_ref, wuk_ref, wuv_ref, c_out, wuk_out, wuv_out,
                   local_sems, send_sems, recv_sems):
    my_x = lax.axis_index("x")
    my_y = lax.axis_index("y")
    my_z = lax.axis_index("z")
    peer = (my_x, my_y, 1 - my_z)

    barrier = pltpu.get_barrier_semaphore()
    pl.semaphore_signal(barrier, inc=1, device_id=peer,
                        device_id_type=pl.DeviceIdType.MESH)
    pl.semaphore_wait(barrier, 1)

    copies = []
    for i, (src, dst) in enumerate(
        [(c_ref, c_out), (wuk_ref, wuk_out), (wuv_ref, wuv_out)]
    ):
        local = pltpu.make_async_copy(src, dst.at[my_z], local_sems.at[i])
        local.start()
        rdma = pltpu.make_async_remote_copy(
            src_ref=src,
            dst_ref=dst.at[my_z],
            send_sem=send_sems.at[i],
            recv_sem=recv_sems.at[i],
            device_id=peer,
            device_id_type=pl.DeviceIdType.MESH,
        )
        rdma.start()
        copies.append((local, rdma))
    for local, rdma in copies:
        local.wait()
        rdma.wait()


def _exchange(c_half, wuk, wuv):
    return pl.pallas_call(
        _exchange_body,
        out_shape=[
            jax.ShapeDtypeStruct((2, M, DCL), F32),
            jax.ShapeDtypeStruct((2, DCL, D), F32),
            jax.ShapeDtypeStruct((2, DCL, D), F32),
        ],
        in_specs=[pl.BlockSpec(memory_space=pltpu.VMEM)] * 3,
        out_specs=[pl.BlockSpec(memory_space=pltpu.VMEM)] * 3,
        scratch_shapes=[
            pltpu.SemaphoreType.DMA((3,)),
            pltpu.SemaphoreType.DMA((3,)),
            pltpu.SemaphoreType.DMA((3,)),
        ],
        compiler_params=pltpu.CompilerParams(collective_id=0),
    )(c_half, wuk, wuv)


def _kv_body(c_ref, wuk_ref, wuv_ref, k_ref, v_ref):
    c0 = c_ref[0]
    c1 = c_ref[1]
    k_ref[:, :] = (jnp.dot(c0, wuk_ref[0], preferred_element_type=F32)
                   + jnp.dot(c1, wuk_ref[1], preferred_element_type=F32))
    v_ref[:, :] = (jnp.dot(c0, wuv_ref[0], preferred_element_type=F32)
                   + jnp.dot(c1, wuv_ref[1], preferred_element_type=F32))


def _kv(c_st, wuk_st, wuv_st, block_n=512):
    return pl.pallas_call(
        _kv_body,
        grid=(D // block_n,),
        in_specs=[
            pl.BlockSpec((2, M, DCL), lambda j: (0, 0, 0)),
            pl.BlockSpec((2, DCL, block_n), lambda j: (0, 0, j)),
            pl.BlockSpec((2, DCL, block_n), lambda j: (0, 0, j)),
        ],
        out_specs=[
            pl.BlockSpec((M, block_n), lambda j: (0, j)),
            pl.BlockSpec((M, block_n), lambda j: (0, j)),
        ],
        out_shape=[
            jax.ShapeDtypeStruct((M, D), F32),
            jax.ShapeDtypeStruct((M, D), F32),
        ],
    )(c_st, wuk_st, wuv_st)


HB = 2


def _attn_body(q_ref, k_ref, v_ref, qr_ref, kr_ref, o_ref):
    dn = (((1,), (1,)), ((), ()))
    kr = kr_ref[:, :]
    for t in range(HB):
        q = q_ref[:, t * Dh:(t + 1) * Dh]
        k = k_ref[:, t * Dh:(t + 1) * Dh]
        v = v_ref[:, t * Dh:(t + 1) * Dh]
        qr = qr_ref[:, t * Dr:(t + 1) * Dr]
        s = lax.dot_general(q, k, dn, preferred_element_type=F32)
        s += lax.dot_general(qr, kr, dn, preferred_element_type=F32)
        s *= SCALE
        m = jnp.max(s, axis=1, keepdims=True)
        p = jnp.exp(s - m)
        p /= jnp.sum(p, axis=1, keepdims=True)
        o_ref[:, t * Dh:(t + 1) * Dh] = jnp.dot(
            p, v, preferred_element_type=F32)


def _attention(q2d, k2d, v2d, qr2d, kr2d):
    return pl.pallas_call(
        _attn_body,
        grid=(B, H // HB),
        in_specs=[
            pl.BlockSpec((S, HB * Dh), lambda b, j: (b, j)),
            pl.BlockSpec((S, HB * Dh), lambda b, j: (b, j)),
            pl.BlockSpec((S, HB * Dh), lambda b, j: (b, j)),
            pl.BlockSpec((S, HB * Dr), lambda b, j: (b, j)),
            pl.BlockSpec((S, Dr), lambda b, j: (b, 0)),
        ],
        out_specs=pl.BlockSpec((S, HB * Dh), lambda b, j: (b, j)),
        out_shape=jax.ShapeDtypeStruct((M, D), F32),
    )(q2d, k2d, v2d, qr2d, kr2d)


def kernel(x, Wdkv, Wuk, Wuv, Wq, Wqr, Wkr, Wo):
    x2d = x.reshape(M, D)
    c_half = _matmul(x2d, Wdkv, DCL)
    c_st, wuk_st, wuv_st = _exchange(c_half, Wuk, Wuv)
    k2d, v2d = _kv(c_st, wuk_st, wuv_st)
    q2d = _matmul(x2d, Wq, 512)
    qr2d = _matmul(x2d, Wqr, 512)
    kr2d = _matmul(x2d, Wkr, 64)
    o2d = _attention(q2d, k2d, v2d, qr2d, kr2d)
    out2d = _matmul(o2d, Wo, 512)
    return out2d.reshape(B, S, D)


# device time: 263957 ns/iter; 1.1299x vs baseline; 1.1299x over previous
import jax
import jax.numpy as jnp
from jax import lax
from jax.experimental import pallas as pl
from jax.experimental.pallas import tpu as pltpu

B, S, H, Dh, Dr = 4, 256, 32, 128, 64
D = 4096
DCL = 128
M = B * S
F32 = jnp.float32
SCALE = float((Dh + Dr) ** -0.5)


def _matmul_body(x_ref, w_ref, o_ref):
    o_ref[:, :] = jnp.dot(x_ref[:, :], w_ref[:, :],
                          preferred_element_type=F32)


def _matmul(x, w, block_n):
    m, k = x.shape
    _, n = w.shape
    return pl.pallas_call(
        _matmul_body,
        grid=(n // block_n,),
        in_specs=[
            pl.BlockSpec((m, k), lambda j: (0, 0)),
            pl.BlockSpec((k, block_n), lambda j: (0, j)),
        ],
        out_specs=pl.BlockSpec((m, block_n), lambda j: (0, j)),
        out_shape=jax.ShapeDtypeStruct((m, n), F32),
        compiler_params=pltpu.CompilerParams(
            vmem_limit_bytes=48 * 1024 * 1024),
    )(x, w)


_QB = 512


def _fused_body(x_ref, wdkv_ref, wuk_ref, wuv_ref, wq_ref,
                q_ref, c_out, wuk_out, wuv_out,
                c_scr, local_sems, send_sems, recv_sems):
    j = pl.program_id(0)
    nj = pl.num_programs(0)
    my_x = lax.axis_index("x")
    my_y = lax.axis_index("y")
    my_z = lax.axis_index("z")
    peer = (my_x, my_y, 1 - my_z)

    def _descriptors():
        out = []
        for i, (src, dst) in enumerate(
            [(c_scr, c_out), (wuk_ref, wuk_out), (wuv_ref, wuv_out)]
        ):
            local = pltpu.make_async_copy(src, dst.at[my_z],
                                          local_sems.at[i])
            rdma = pltpu.make_async_remote_copy(
                src_ref=src,
                dst_ref=dst.at[my_z],
                send_sem=send_sems.at[i],
                recv_sem=recv_sems.at[i],
                device_id=peer,
                device_id_type=pl.DeviceIdType.MESH,
            )
            out.append((local, rdma))
        return out

    @pl.when(j == 0)
    def _():
        barrier = pltpu.get_barrier_semaphore()
        pl.semaphore_signal(barrier, inc=1, device_id=peer,
                            device_id_type=pl.DeviceIdType.MESH)
        pl.semaphore_wait(barrier, 1)

        c_scr[:, :] = jnp.dot(x_ref[:, :], wdkv_ref[:, :],
                              preferred_element_type=F32)
        for local, rdma in _descriptors():
            local.start()
            rdma.start()

    q_ref[:, :] = jnp.dot(x_ref[:, :], wq_ref[:, :],
                          preferred_element_type=F32)

    @pl.when(j == nj - 1)
    def _():
        for local, rdma in _descriptors():
            local.wait()
            rdma.wait()


def _fused_q_exchange(x2d, wdkv, wuk, wuv, wq):
    return pl.pallas_call(
        _fused_body,
        grid=(D // _QB,),
        in_specs=[
            pl.BlockSpec((M, D), lambda j: (0, 0)),
            pl.BlockSpec((D, DCL), lambda j: (0, 0)),
            pl.BlockSpec((DCL, D), lambda j: (0, 0)),
            pl.BlockSpec((DCL, D), lambda j: (0, 0)),
            pl.BlockSpec((D, _QB), lambda j: (0, j)),
        ],
        out_specs=[
            pl.BlockSpec((M, _QB), lambda j: (0, j)),
            pl.BlockSpec((2, M, DCL), lambda j: (0, 0, 0)),
            pl.BlockSpec((2, DCL, D), lambda j: (0, 0, 0)),
            pl.BlockSpec((2, DCL, D), lambda j: (0, 0, 0)),
        ],
        out_shape=[
            jax.ShapeDtypeStruct((M, D), F32),
            jax.ShapeDtypeStruct((2, M, DCL), F32),
            jax.ShapeDtypeStruct((2, DCL, D), F32),
            jax.ShapeDtypeStruct((2, DCL, D), F32),
        ],
        scratch_shapes=[
            pltpu.VMEM((M, DCL), F32),
            pltpu.SemaphoreType.DMA((3,)),
            pltpu.SemaphoreType.DMA((3,)),
            pltpu.SemaphoreType.DMA((3,)),
        ],
        compiler_params=pltpu.CompilerParams(
            collective_id=0, vmem_limit_bytes=96 * 1024 * 1024),
    )(x2d, wdkv, wuk, wuv, wq)


def _kv_body(c_ref, wuk_ref, wuv_ref, k_ref, v_ref):
    c0 = c_ref[0]
    c1 = c_ref[1]
    k_ref[:, :] = (jnp.dot(c0, wuk_ref[0], preferred_element_type=F32)
                   + jnp.dot(c1, wuk_ref[1], preferred_element_type=F32))
    v_ref[:, :] = (jnp.dot(c0, wuv_ref[0], preferred_element_type=F32)
                   + jnp.dot(c1, wuv_ref[1], preferred_element_type=F32))


def _kv(c_st, wuk_st, wuv_st, block_n=512):
    return pl.pallas_call(
        _kv_body,
        grid=(D // block_n,),
        in_specs=[
            pl.BlockSpec((2, M, DCL), lambda j: (0, 0, 0)),
            pl.BlockSpec((2, DCL, block_n), lambda j: (0, 0, j)),
            pl.BlockSpec((2, DCL, block_n), lambda j: (0, 0, j)),
        ],
        out_specs=[
            pl.BlockSpec((M, block_n), lambda j: (0, j)),
            pl.BlockSpec((M, block_n), lambda j: (0, j)),
        ],
        out_shape=[
            jax.ShapeDtypeStruct((M, D), F32),
            jax.ShapeDtypeStruct((M, D), F32),
        ],
    )(c_st, wuk_st, wuv_st)


HB = 2


def _attn_body(q_ref, k_ref, v_ref, qr_ref, kr_ref, o_ref):
    dn = (((1,), (1,)), ((), ()))
    kr = kr_ref[:, :]
    for t in range(HB):
        q = q_ref[:, t * Dh:(t + 1) * Dh]
        k = k_ref[:, t * Dh:(t + 1) * Dh]
        v = v_ref[:, t * Dh:(t + 1) * Dh]
        qr = qr_ref[:, t * Dr:(t + 1) * Dr]
        s = lax.dot_general(q, k, dn, preferred_element_type=F32)
        s += lax.dot_general(qr, kr, dn, preferred_element_type=F32)
        s *= SCALE
        m = jnp.max(s, axis=1, keepdims=True)
        p = jnp.exp(s - m)
        p /= jnp.sum(p, axis=1, keepdims=True)
        o_ref[:, t * Dh:(t + 1) * Dh] = jnp.dot(
            p, v, preferred_element_type=F32)


def _attention(q2d, k2d, v2d, qr2d, kr2d):
    return pl.pallas_call(
        _attn_body,
        grid=(B, H // HB),
        in_specs=[
            pl.BlockSpec((S, HB * Dh), lambda b, j: (b, j)),
            pl.BlockSpec((S, HB * Dh), lambda b, j: (b, j)),
            pl.BlockSpec((S, HB * Dh), lambda b, j: (b, j)),
            pl.BlockSpec((S, HB * Dr), lambda b, j: (b, j)),
            pl.BlockSpec((S, Dr), lambda b, j: (b, 0)),
        ],
        out_specs=pl.BlockSpec((S, HB * Dh), lambda b, j: (b, j)),
        out_shape=jax.ShapeDtypeStruct((M, D), F32),
    )(q2d, k2d, v2d, qr2d, kr2d)


def kernel(x, Wdkv, Wuk, Wuv, Wq, Wqr, Wkr, Wo):
    x2d = x.reshape(M, D)
    q2d, c_st, wuk_st, wuv_st = _fused_q_exchange(x2d, Wdkv, Wuk, Wuv, Wq)
    k2d, v2d = _kv(c_st, wuk_st, wuv_st)
    qr2d = _matmul(x2d, Wqr, 512)
    kr2d = _matmul(x2d, Wkr, 64)
    o2d = _attention(q2d, k2d, v2d, qr2d, kr2d)
    out2d = _matmul(o2d, Wo, 512)
    return out2d.reshape(B, S, D)


# device time: 263864 ns/iter; 1.1303x vs baseline; 1.0004x over previous
import jax
import jax.numpy as jnp
from jax import lax
from jax.experimental import pallas as pl
from jax.experimental.pallas import tpu as pltpu

B, S, H, Dh, Dr = 4, 256, 32, 128, 64
D = 4096
DCL = 128
M = B * S
F32 = jnp.float32
SCALE = float((Dh + Dr) ** -0.5)


def _matmul_body(x_ref, w_ref, o_ref, *, cast=False):
    xv, wv = x_ref[:, :], w_ref[:, :]
    if cast:
        xv = xv.astype(jnp.bfloat16)
        wv = wv.astype(jnp.bfloat16)
    o_ref[:, :] = jnp.dot(xv, wv, preferred_element_type=F32)


def _matmul(x, w, block_n, cast=False):
    import functools
    m, k = x.shape
    _, n = w.shape
    return pl.pallas_call(
        functools.partial(_matmul_body, cast=cast),
        grid=(n // block_n,),
        in_specs=[
            pl.BlockSpec((m, k), lambda j: (0, 0)),
            pl.BlockSpec((k, block_n), lambda j: (0, j)),
        ],
        out_specs=pl.BlockSpec((m, block_n), lambda j: (0, j)),
        out_shape=jax.ShapeDtypeStruct((m, n), F32),
        compiler_params=pltpu.CompilerParams(
            vmem_limit_bytes=48 * 1024 * 1024),
    )(x, w)


_QB = 512


def _fused_body(x_ref, wdkv_ref, wuk_ref, wuv_ref, wq_ref,
                q_ref, c_out, wuk_out, wuv_out,
                c_scr, local_sems, send_sems, recv_sems):
    j = pl.program_id(0)
    nj = pl.num_programs(0)
    my_x = lax.axis_index("x")
    my_y = lax.axis_index("y")
    my_z = lax.axis_index("z")
    peer = (my_x, my_y, 1 - my_z)

    def _descriptors():
        out = []
        for i, (src, dst) in enumerate(
            [(c_scr, c_out), (wuk_ref, wuk_out), (wuv_ref, wuv_out)]
        ):
            local = pltpu.make_async_copy(src, dst.at[my_z],
                                          local_sems.at[i])
            rdma = pltpu.make_async_remote_copy(
                src_ref=src,
                dst_ref=dst.at[my_z],
                send_sem=send_sems.at[i],
                recv_sem=recv_sems.at[i],
                device_id=peer,
                device_id_type=pl.DeviceIdType.MESH,
            )
            out.append((local, rdma))
        return out

    @pl.when(j == 0)
    def _():
        barrier = pltpu.get_barrier_semaphore()
        pl.semaphore_signal(barrier, inc=1, device_id=peer,
                            device_id_type=pl.DeviceIdType.MESH)
        pl.semaphore_wait(barrier, 1)

        c_scr[:, :] = jnp.dot(x_ref[:, :], wdkv_ref[:, :],
                              preferred_element_type=F32)
        for local, rdma in _descriptors():
            local.start()
            rdma.start()

    q_ref[:, :] = jnp.dot(x_ref[:, :].astype(jnp.bfloat16),
                          wq_ref[:, :].astype(jnp.bfloat16),
                          preferred_element_type=F32)

    @pl.when(j == nj - 1)
    def _():
        for local, rdma in _descriptors():
            local.wait()
            rdma.wait()


def _fused_q_exchange(x2d, wdkv, wuk, wuv, wq):
    return pl.pallas_call(
        _fused_body,
        grid=(D // _QB,),
        in_specs=[
            pl.BlockSpec((M, D), lambda j: (0, 0)),
            pl.BlockSpec((D, DCL), lambda j: (0, 0)),
            pl.BlockSpec((DCL, D), lambda j: (0, 0)),
            pl.BlockSpec((DCL, D), lambda j: (0, 0)),
            pl.BlockSpec((D, _QB), lambda j: (0, j)),
        ],
        out_specs=[
            pl.BlockSpec((M, _QB), lambda j: (0, j)),
            pl.BlockSpec((2, M, DCL), lambda j: (0, 0, 0)),
            pl.BlockSpec((2, DCL, D), lambda j: (0, 0, 0)),
            pl.BlockSpec((2, DCL, D), lambda j: (0, 0, 0)),
        ],
        out_shape=[
            jax.ShapeDtypeStruct((M, D), F32),
            jax.ShapeDtypeStruct((2, M, DCL), F32),
            jax.ShapeDtypeStruct((2, DCL, D), F32),
            jax.ShapeDtypeStruct((2, DCL, D), F32),
        ],
        scratch_shapes=[
            pltpu.VMEM((M, DCL), F32),
            pltpu.SemaphoreType.DMA((3,)),
            pltpu.SemaphoreType.DMA((3,)),
            pltpu.SemaphoreType.DMA((3,)),
        ],
        compiler_params=pltpu.CompilerParams(
            collective_id=0, vmem_limit_bytes=96 * 1024 * 1024),
    )(x2d, wdkv, wuk, wuv, wq)


def _kv_body(c_ref, wuk_ref, wuv_ref, k_ref, v_ref):
    c0 = c_ref[0]
    c1 = c_ref[1]
    k_ref[:, :] = (jnp.dot(c0, wuk_ref[0], preferred_element_type=F32)
                   + jnp.dot(c1, wuk_ref[1], preferred_element_type=F32))
    v_ref[:, :] = (jnp.dot(c0, wuv_ref[0], preferred_element_type=F32)
                   + jnp.dot(c1, wuv_ref[1], preferred_element_type=F32))


def _kv(c_st, wuk_st, wuv_st, block_n=512):
    return pl.pallas_call(
        _kv_body,
        grid=(D // block_n,),
        in_specs=[
            pl.BlockSpec((2, M, DCL), lambda j: (0, 0, 0)),
            pl.BlockSpec((2, DCL, block_n), lambda j: (0, 0, j)),
            pl.BlockSpec((2, DCL, block_n), lambda j: (0, 0, j)),
        ],
        out_specs=[
            pl.BlockSpec((M, block_n), lambda j: (0, j)),
            pl.BlockSpec((M, block_n), lambda j: (0, j)),
        ],
        out_shape=[
            jax.ShapeDtypeStruct((M, D), F32),
            jax.ShapeDtypeStruct((M, D), F32),
        ],
    )(c_st, wuk_st, wuv_st)


HB = 2


def _attn_body(q_ref, k_ref, v_ref, qr_ref, kr_ref, o_ref):
    dn = (((1,), (1,)), ((), ()))
    kr = kr_ref[:, :]
    for t in range(HB):
        q = q_ref[:, t * Dh:(t + 1) * Dh]
        k = k_ref[:, t * Dh:(t + 1) * Dh]
        v = v_ref[:, t * Dh:(t + 1) * Dh]
        qr = qr_ref[:, t * Dr:(t + 1) * Dr]
        s = lax.dot_general(q, k, dn, preferred_element_type=F32)
        s += lax.dot_general(qr, kr, dn, preferred_element_type=F32)
        s *= SCALE
        m = jnp.max(s, axis=1, keepdims=True)
        p = jnp.exp(s - m)
        p /= jnp.sum(p, axis=1, keepdims=True)
        o_ref[:, t * Dh:(t + 1) * Dh] = jnp.dot(
            p, v, preferred_element_type=F32)


def _attention(q2d, k2d, v2d, qr2d, kr2d):
    return pl.pallas_call(
        _attn_body,
        grid=(B, H // HB),
        in_specs=[
            pl.BlockSpec((S, HB * Dh), lambda b, j: (b, j)),
            pl.BlockSpec((S, HB * Dh), lambda b, j: (b, j)),
            pl.BlockSpec((S, HB * Dh), lambda b, j: (b, j)),
            pl.BlockSpec((S, HB * Dr), lambda b, j: (b, j)),
            pl.BlockSpec((S, Dr), lambda b, j: (b, 0)),
        ],
        out_specs=pl.BlockSpec((S, HB * Dh), lambda b, j: (b, j)),
        out_shape=jax.ShapeDtypeStruct((M, D), F32),
    )(q2d, k2d, v2d, qr2d, kr2d)


def kernel(x, Wdkv, Wuk, Wuv, Wq, Wqr, Wkr, Wo):
    x2d = x.reshape(M, D)
    q2d, c_st, wuk_st, wuv_st = _fused_q_exchange(x2d, Wdkv, Wuk, Wuv, Wq)
    k2d, v2d = _kv(c_st, wuk_st, wuv_st)
    qr2d = _matmul(x2d, Wqr, 512, cast=True)
    kr2d = _matmul(x2d, Wkr, 64)
    o2d = _attention(q2d, k2d, v2d, qr2d, kr2d)
    out2d = _matmul(o2d, Wo, 512, cast=True)
    return out2d.reshape(B, S, D)


# device time: 227061 ns/iter; 1.3135x vs baseline; 1.1621x over previous
import jax
import jax.numpy as jnp
from jax import lax
from jax.experimental import pallas as pl
from jax.experimental.pallas import tpu as pltpu

B, S, H, Dh, Dr = 4, 256, 32, 128, 64
D = 4096
DCL = 128
M = B * S
F32 = jnp.float32
SCALE = float((Dh + Dr) ** -0.5)


def _matmul_body(x_ref, w_ref, o_ref):
    o_ref[:, :] = jnp.dot(x_ref[:, :], w_ref[:, :],
                          preferred_element_type=F32)


def _matmul(x, w, block_n):
    m, k = x.shape
    _, n = w.shape
    return pl.pallas_call(
        _matmul_body,
        grid=(n // block_n,),
        in_specs=[
            pl.BlockSpec((m, k), lambda j: (0, 0)),
            pl.BlockSpec((k, block_n), lambda j: (0, j)),
        ],
        out_specs=pl.BlockSpec((m, block_n), lambda j: (0, j)),
        out_shape=jax.ShapeDtypeStruct((m, n), F32),
        compiler_params=pltpu.CompilerParams(
            vmem_limit_bytes=48 * 1024 * 1024),
    )(x, w)


_QB = 512


def _fused_body(x_ref, wdkv_ref, wuk_ref, wuv_ref, wq_ref,
                q_ref, c_out, wuk_out, wuv_out,
                c_scr, local_sems, send_sems, recv_sems):
    j = pl.program_id(0)
    nj = pl.num_programs(0)
    my_x = lax.axis_index("x")
    my_y = lax.axis_index("y")
    my_z = lax.axis_index("z")
    peer = (my_x, my_y, 1 - my_z)

    def _descriptors():
        out = []
        for i, (src, dst) in enumerate(
            [(c_scr, c_out), (wuk_ref, wuk_out), (wuv_ref, wuv_out)]
        ):
            local = pltpu.make_async_copy(src, dst.at[my_z],
                                          local_sems.at[i])
            rdma = pltpu.make_async_remote_copy(
                src_ref=src,
                dst_ref=dst.at[my_z],
                send_sem=send_sems.at[i],
                recv_sem=recv_sems.at[i],
                device_id=peer,
                device_id_type=pl.DeviceIdType.MESH,
            )
            out.append((local, rdma))
        return out

    @pl.when(j == 0)
    def _():
        barrier = pltpu.get_barrier_semaphore()
        pl.semaphore_signal(barrier, inc=1, device_id=peer,
                            device_id_type=pl.DeviceIdType.MESH)
        pl.semaphore_wait(barrier, 1)

        c_scr[:, :] = jnp.dot(x_ref[:, :], wdkv_ref[:, :],
                              preferred_element_type=F32)
        for local, rdma in _descriptors():
            local.start()
            rdma.start()

    q_ref[:, :] = jnp.dot(x_ref[:, :], wq_ref[:, :],
                          preferred_element_type=F32) * SCALE

    @pl.when(j == nj - 1)
    def _():
        for local, rdma in _descriptors():
            local.wait()
            rdma.wait()


def _fused_q_exchange(x2d, wdkv, wuk, wuv, wq):
    return pl.pallas_call(
        _fused_body,
        grid=(D // _QB,),
        in_specs=[
            pl.BlockSpec((M, D), lambda j: (0, 0)),
            pl.BlockSpec((D, DCL), lambda j: (0, 0)),
            pl.BlockSpec((DCL, D), lambda j: (0, 0)),
            pl.BlockSpec((DCL, D), lambda j: (0, 0)),
            pl.BlockSpec((D, _QB), lambda j: (0, j)),
        ],
        out_specs=[
            pl.BlockSpec((M, _QB), lambda j: (0, j)),
            pl.BlockSpec((2, M, DCL), lambda j: (0, 0, 0)),
            pl.BlockSpec((2, DCL, D), lambda j: (0, 0, 0)),
            pl.BlockSpec((2, DCL, D), lambda j: (0, 0, 0)),
        ],
        out_shape=[
            jax.ShapeDtypeStruct((M, D), F32),
            jax.ShapeDtypeStruct((2, M, DCL), F32),
            jax.ShapeDtypeStruct((2, DCL, D), F32),
            jax.ShapeDtypeStruct((2, DCL, D), F32),
        ],
        scratch_shapes=[
            pltpu.VMEM((M, DCL), F32),
            pltpu.SemaphoreType.DMA((3,)),
            pltpu.SemaphoreType.DMA((3,)),
            pltpu.SemaphoreType.DMA((3,)),
        ],
        compiler_params=pltpu.CompilerParams(
            collective_id=0, vmem_limit_bytes=96 * 1024 * 1024),
    )(x2d, wdkv, wuk, wuv, wq)


def _kv_body(c_ref, wuk_ref, wuv_ref, k_ref, v_ref):
    c0 = c_ref[0]
    c1 = c_ref[1]
    k_ref[:, :] = (jnp.dot(c0, wuk_ref[0], preferred_element_type=F32)
                   + jnp.dot(c1, wuk_ref[1], preferred_element_type=F32))
    v_ref[:, :] = (jnp.dot(c0, wuv_ref[0], preferred_element_type=F32)
                   + jnp.dot(c1, wuv_ref[1], preferred_element_type=F32))


def _kv(c_st, wuk_st, wuv_st, block_n=512):
    return pl.pallas_call(
        _kv_body,
        grid=(D // block_n,),
        in_specs=[
            pl.BlockSpec((2, M, DCL), lambda j: (0, 0, 0)),
            pl.BlockSpec((2, DCL, block_n), lambda j: (0, 0, j)),
            pl.BlockSpec((2, DCL, block_n), lambda j: (0, 0, j)),
        ],
        out_specs=[
            pl.BlockSpec((M, block_n), lambda j: (0, j)),
            pl.BlockSpec((M, block_n), lambda j: (0, j)),
        ],
        out_shape=[
            jax.ShapeDtypeStruct((M, D), F32),
            jax.ShapeDtypeStruct((M, D), F32),
        ],
    )(c_st, wuk_st, wuv_st)


_QRB = 512


def _qrkr_body(x_ref, wqr_ref, wkr_ref, qr_ref, kr_ref):
    j = pl.program_id(0)
    qr_ref[:, :] = jnp.dot(x_ref[:, :], wqr_ref[:, :],
                           preferred_element_type=F32) * SCALE

    @pl.when(j == 0)
    def _():
        kr_ref[:, :] = jnp.dot(x_ref[:, :], wkr_ref[:, :],
                               preferred_element_type=F32)


def _qrkr(x2d, wqr, wkr):
    return pl.pallas_call(
        _qrkr_body,
        grid=(H * Dr // _QRB,),
        in_specs=[
            pl.BlockSpec((M, D), lambda j: (0, 0)),
            pl.BlockSpec((D, _QRB), lambda j: (0, j)),
            pl.BlockSpec((D, Dr), lambda j: (0, 0)),
        ],
        out_specs=[
            pl.BlockSpec((M, _QRB), lambda j: (0, j)),
            pl.BlockSpec((M, Dr), lambda j: (0, 0)),
        ],
        out_shape=[
            jax.ShapeDtypeStruct((M, H * Dr), F32),
            jax.ShapeDtypeStruct((M, Dr), F32),
        ],
        compiler_params=pltpu.CompilerParams(
            vmem_limit_bytes=48 * 1024 * 1024),
    )(x2d, wqr, wkr)


HB = 4


def _attn_body(q_ref, k_ref, v_ref, qr_ref, kr_ref, o_ref):
    dn = (((1,), (1,)), ((), ()))
    kr = kr_ref[:, :]
    for t in range(HB):
        q = q_ref[:, t * Dh:(t + 1) * Dh]
        k = k_ref[:, t * Dh:(t + 1) * Dh]
        v = v_ref[:, t * Dh:(t + 1) * Dh]
        qr = qr_ref[:, t * Dr:(t + 1) * Dr]
        s = lax.dot_general(q, k, dn, preferred_element_type=F32)
        s += lax.dot_general(qr, kr, dn, preferred_element_type=F32)
        p = jnp.exp(s)
        r = 1.0 / jnp.sum(p, axis=1, keepdims=True)
        o_ref[:, t * Dh:(t + 1) * Dh] = jnp.dot(
            p, v, preferred_element_type=F32) * r


def _attention(q2d, k2d, v2d, qr2d, kr2d):
    return pl.pallas_call(
        _attn_body,
        grid=(B, H // HB),
        in_specs=[
            pl.BlockSpec((S, HB * Dh), lambda b, j: (b, j)),
            pl.BlockSpec((S, HB * Dh), lambda b, j: (b, j)),
            pl.BlockSpec((S, HB * Dh), lambda b, j: (b, j)),
            pl.BlockSpec((S, HB * Dr), lambda b, j: (b, j)),
            pl.BlockSpec((S, Dr), lambda b, j: (b, 0)),
        ],
        out_specs=pl.BlockSpec((S, HB * Dh), lambda b, j: (b, j)),
        out_shape=jax.ShapeDtypeStruct((M, D), F32),
    )(q2d, k2d, v2d, qr2d, kr2d)


def kernel(x, Wdkv, Wuk, Wuv, Wq, Wqr, Wkr, Wo):
    x2d = x.reshape(M, D)
    q2d, c_st, wuk_st, wuv_st = _fused_q_exchange(x2d, Wdkv, Wuk, Wuv, Wq)
    k2d, v2d = _kv(c_st, wuk_st, wuv_st)
    qr2d, kr2d = _qrkr(x2d, Wqr, Wkr)
    o2d = _attention(q2d, k2d, v2d, qr2d, kr2d)
    out2d = _matmul(o2d, Wo, 512)
    return out2d.reshape(B, S, D)


# device time: 206996 ns/iter; 1.4408x vs baseline; 1.0969x over previous
import jax
import jax.numpy as jnp
from jax import lax
from jax.experimental import pallas as pl
from jax.experimental.pallas import tpu as pltpu

B, S, H, Dh, Dr = 4, 256, 32, 128, 64
D = 4096
DCL = 128
M = B * S
F32 = jnp.float32
SCALE = float((Dh + Dr) ** -0.5)


def _matmul_body(x_ref, w_ref, o_ref):
    o_ref[:, :] = jnp.dot(x_ref[:, :], w_ref[:, :],
                          preferred_element_type=F32)


def _matmul(x, w, block_n):
    m, k = x.shape
    _, n = w.shape
    return pl.pallas_call(
        _matmul_body,
        grid=(n // block_n,),
        in_specs=[
            pl.BlockSpec((m, k), lambda j: (0, 0)),
            pl.BlockSpec((k, block_n), lambda j: (0, j)),
        ],
        out_specs=pl.BlockSpec((m, block_n), lambda j: (0, j)),
        out_shape=jax.ShapeDtypeStruct((m, n), F32),
        compiler_params=pltpu.CompilerParams(
            vmem_limit_bytes=48 * 1024 * 1024),
    )(x, w)


_QB = 256
_NJ_Q = D // _QB
_NJ = _NJ_Q + H * Dr // _QB


def _fused_body(x_ref, wdkv_ref, wuk_ref, wuv_ref, wq_ref, wqr_ref, wkr_ref,
                q_ref, qr_ref, kr_ref, c_out, wuk_out, wuv_out,
                c_scr, local_sems, send_sems, recv_sems):
    j = pl.program_id(0)
    my_x = lax.axis_index("x")
    my_y = lax.axis_index("y")
    my_z = lax.axis_index("z")
    peer = (my_x, my_y, 1 - my_z)

    def _descriptors():
        out = []
        for i, (src, dst) in enumerate(
            [(c_scr, c_out), (wuk_ref, wuk_out), (wuv_ref, wuv_out)]
        ):
            local = pltpu.make_async_copy(src, dst.at[my_z],
                                          local_sems.at[i])
            rdma = pltpu.make_async_remote_copy(
                src_ref=src,
                dst_ref=dst.at[my_z],
                send_sem=send_sems.at[i],
                recv_sem=recv_sems.at[i],
                device_id=peer,
                device_id_type=pl.DeviceIdType.MESH,
            )
            out.append((local, rdma))
        return out

    @pl.when(j == 0)
    def _():
        barrier = pltpu.get_barrier_semaphore()
        pl.semaphore_signal(barrier, inc=1, device_id=peer,
                            device_id_type=pl.DeviceIdType.MESH)
        pl.semaphore_wait(barrier, 1)

        c_scr[:, :] = jnp.dot(x_ref[:, :], wdkv_ref[:, :],
                              preferred_element_type=F32)
        for local, rdma in _descriptors():
            local.start()
            rdma.start()

    @pl.when(j < _NJ_Q)
    def _():
        q_ref[:, :] = jnp.dot(x_ref[:, :], wq_ref[:, :],
                              preferred_element_type=F32) * SCALE

    @pl.when(j >= _NJ_Q)
    def _():
        qr_ref[:, :] = jnp.dot(x_ref[:, :], wqr_ref[:, :],
                               preferred_element_type=F32) * SCALE

    @pl.when(j == _NJ_Q)
    def _():
        kr_ref[:, :] = jnp.dot(x_ref[:, :], wkr_ref[:, :],
                               preferred_element_type=F32)

    @pl.when(j == _NJ - 1)
    def _():
        for local, rdma in _descriptors():
            local.wait()
            rdma.wait()


def _fused_q_exchange(x2d, wdkv, wuk, wuv, wq, wqr, wkr):
    return pl.pallas_call(
        _fused_body,
        grid=(_NJ,),
        in_specs=[
            pl.BlockSpec((M, D), lambda j: (0, 0)),
            pl.BlockSpec((D, DCL), lambda j: (0, 0)),
            pl.BlockSpec((DCL, D), lambda j: (0, 0)),
            pl.BlockSpec((DCL, D), lambda j: (0, 0)),
            pl.BlockSpec((D, _QB),
                         lambda j: (0, jnp.minimum(j, _NJ_Q - 1))),
            pl.BlockSpec((D, _QB),
                         lambda j: (0, jnp.maximum(j - _NJ_Q, 0))),
            pl.BlockSpec((D, Dr), lambda j: (0, 0)),
        ],
        out_specs=[
            pl.BlockSpec((M, _QB),
                         lambda j: (0, jnp.minimum(j, _NJ_Q - 1))),
            pl.BlockSpec((M, _QB),
                         lambda j: (0, jnp.maximum(j - _NJ_Q, 0))),
            pl.BlockSpec((M, Dr), lambda j: (0, 0)),
            pl.BlockSpec((2, M, DCL), lambda j: (0, 0, 0)),
            pl.BlockSpec((2, DCL, D), lambda j: (0, 0, 0)),
            pl.BlockSpec((2, DCL, D), lambda j: (0, 0, 0)),
        ],
        out_shape=[
            jax.ShapeDtypeStruct((M, D), F32),
            jax.ShapeDtypeStruct((M, H * Dr), F32),
            jax.ShapeDtypeStruct((M, Dr), F32),
            jax.ShapeDtypeStruct((2, M, DCL), F32),
            jax.ShapeDtypeStruct((2, DCL, D), F32),
            jax.ShapeDtypeStruct((2, DCL, D), F32),
        ],
        scratch_shapes=[
            pltpu.VMEM((M, DCL), F32),
            pltpu.SemaphoreType.DMA((3,)),
            pltpu.SemaphoreType.DMA((3,)),
            pltpu.SemaphoreType.DMA((3,)),
        ],
        compiler_params=pltpu.CompilerParams(
            collective_id=0, vmem_limit_bytes=100 * 1024 * 1024),
    )(x2d, wdkv, wuk, wuv, wq, wqr, wkr)


HB = 4


def _attn_body(q_ref, qr_ref, kr_ref, c_ref, wuk_ref, wuv_ref, o_ref):
    dn = (((1,), (1,)), ((), ()))
    c0 = c_ref[0]
    c1 = c_ref[1]
    kh = (jnp.dot(c0, wuk_ref[0], preferred_element_type=F32)
          + jnp.dot(c1, wuk_ref[1], preferred_element_type=F32))
    vh = (jnp.dot(c0, wuv_ref[0], preferred_element_type=F32)
          + jnp.dot(c1, wuv_ref[1], preferred_element_type=F32))
    kr = kr_ref[:, :]
    for t in range(HB):
        q = q_ref[:, t * Dh:(t + 1) * Dh]
        k = kh[:, t * Dh:(t + 1) * Dh]
        v = vh[:, t * Dh:(t + 1) * Dh]
        qr = qr_ref[:, t * Dr:(t + 1) * Dr]
        s = lax.dot_general(q, k, dn, preferred_element_type=F32)
        s += lax.dot_general(qr, kr, dn, preferred_element_type=F32)
        p = jnp.exp(s)
        r = 1.0 / jnp.sum(p, axis=1, keepdims=True)
        o_ref[:, t * Dh:(t + 1) * Dh] = jnp.dot(
            p, v, preferred_element_type=F32) * r


def _attention(q2d, qr2d, kr2d, c_st, wuk_st, wuv_st):
    return pl.pallas_call(
        _attn_body,
        grid=(H // HB, B),
        in_specs=[
            pl.BlockSpec((S, HB * Dh), lambda j, b: (b, j)),
            pl.BlockSpec((S, HB * Dr), lambda j, b: (b, j)),
            pl.BlockSpec((S, Dr), lambda j, b: (b, 0)),
            pl.BlockSpec((2, S, DCL), lambda j, b: (0, b, 0)),
            pl.BlockSpec((2, DCL, HB * Dh), lambda j, b: (0, 0, j)),
            pl.BlockSpec((2, DCL, HB * Dh), lambda j, b: (0, 0, j)),
        ],
        out_specs=pl.BlockSpec((S, HB * Dh), lambda j, b: (b, j)),
        out_shape=jax.ShapeDtypeStruct((M, D), F32),
    )(q2d, qr2d, kr2d, c_st, wuk_st, wuv_st)


def kernel(x, Wdkv, Wuk, Wuv, Wq, Wqr, Wkr, Wo):
    x2d = x.reshape(M, D)
    q2d, qr2d, kr2d, c_st, wuk_st, wuv_st = _fused_q_exchange(
        x2d, Wdkv, Wuk, Wuv, Wq, Wqr, Wkr)
    o2d = _attention(q2d, qr2d, kr2d, c_st, wuk_st, wuv_st)
    out2d = _matmul(o2d, Wo, 512)
    return out2d.reshape(B, S, D)


# device time: 197508 ns/iter; 1.5101x vs baseline; 1.0480x over previous
import jax
import jax.numpy as jnp
from jax import lax
from jax.experimental import pallas as pl
from jax.experimental.pallas import tpu as pltpu

B, S, H, Dh, Dr = 4, 256, 32, 128, 64
D = 4096
DCL = 128
M = B * S
F32 = jnp.float32
SCALE = float((Dh + Dr) ** -0.5)


def _matmul_body(x_ref, w_ref, o_ref):
    o_ref[:, :] = jnp.dot(x_ref[:, :], w_ref[:, :],
                          preferred_element_type=F32)


def _matmul(x, w, block_n):
    m, k = x.shape
    _, n = w.shape
    return pl.pallas_call(
        _matmul_body,
        grid=(n // block_n,),
        in_specs=[
            pl.BlockSpec((m, k), lambda j: (0, 0)),
            pl.BlockSpec((k, block_n), lambda j: (0, j)),
        ],
        out_specs=pl.BlockSpec((m, block_n), lambda j: (0, j)),
        out_shape=jax.ShapeDtypeStruct((m, n), F32),
        compiler_params=pltpu.CompilerParams(
            vmem_limit_bytes=48 * 1024 * 1024),
    )(x, w)


_QB = 256
_NJ_Q = D // _QB
_NJ = _NJ_Q + H * Dr // _QB


def _fused_body(x_ref, wdkv_ref, wuk_ref, wuv_ref, wq_ref, wqr_ref, wkr_ref,
                q_ref, qr_ref, kr_ref, c_out, wuk_out, wuv_out,
                c_scr, local_sems, send_sems, recv_sems):
    j = pl.program_id(0)
    my_x = lax.axis_index("x")
    my_y = lax.axis_index("y")
    my_z = lax.axis_index("z")
    peer = (my_x, my_y, 1 - my_z)

    def _descriptors():
        out = []
        for i, (src, dst) in enumerate(
            [(c_scr, c_out), (wuk_ref, wuk_out), (wuv_ref, wuv_out)]
        ):
            local = pltpu.make_async_copy(src, dst.at[my_z],
                                          local_sems.at[i])
            rdma = pltpu.make_async_remote_copy(
                src_ref=src,
                dst_ref=dst.at[my_z],
                send_sem=send_sems.at[i],
                recv_sem=recv_sems.at[i],
                device_id=peer,
                device_id_type=pl.DeviceIdType.MESH,
            )
            out.append((local, rdma))
        return out

    @pl.when(j == 0)
    def _():
        barrier = pltpu.get_barrier_semaphore()
        pl.semaphore_signal(barrier, inc=1, device_id=peer,
                            device_id_type=pl.DeviceIdType.MESH)
        pl.semaphore_wait(barrier, 1)

        c_scr[:, :] = jnp.dot(x_ref[:, :], wdkv_ref[:, :],
                              preferred_element_type=F32)
        for local, rdma in _descriptors():
            local.start()
            rdma.start()

    @pl.when(j < _NJ_Q)
    def _():
        q_ref[:, :] = jnp.dot(x_ref[:, :], wq_ref[:, :],
                              preferred_element_type=F32) * SCALE

    @pl.when(j >= _NJ_Q)
    def _():
        qr_ref[:, :] = jnp.dot(x_ref[:, :], wqr_ref[:, :],
                               preferred_element_type=F32) * SCALE

    @pl.when(j == _NJ_Q)
    def _():
        kr_ref[:, :] = jnp.dot(x_ref[:, :], wkr_ref[:, :],
                               preferred_element_type=F32)

    @pl.when(j == _NJ - 1)
    def _():
        for local, rdma in _descriptors():
            local.wait()
            rdma.wait()


def _fused_q_exchange(x2d, wdkv, wuk, wuv, wq, wqr, wkr):
    return pl.pallas_call(
        _fused_body,
        grid=(_NJ,),
        in_specs=[
            pl.BlockSpec((M, D), lambda j: (0, 0)),
            pl.BlockSpec((D, DCL), lambda j: (0, 0)),
            pl.BlockSpec((DCL, D), lambda j: (0, 0)),
            pl.BlockSpec((DCL, D), lambda j: (0, 0)),
            pl.BlockSpec((D, _QB),
                         lambda j: (0, jnp.minimum(j, _NJ_Q - 1))),
            pl.BlockSpec((D, _QB),
                         lambda j: (0, jnp.maximum(j - _NJ_Q, 0))),
            pl.BlockSpec((D, Dr), lambda j: (0, 0)),
        ],
        out_specs=[
            pl.BlockSpec((M, _QB),
                         lambda j: (0, jnp.minimum(j, _NJ_Q - 1))),
            pl.BlockSpec((M, _QB),
                         lambda j: (0, jnp.maximum(j - _NJ_Q, 0))),
            pl.BlockSpec((M, Dr), lambda j: (0, 0)),
            pl.BlockSpec((2, M, DCL), lambda j: (0, 0, 0)),
            pl.BlockSpec((2, DCL, D), lambda j: (0, 0, 0)),
            pl.BlockSpec((2, DCL, D), lambda j: (0, 0, 0)),
        ],
        out_shape=[
            jax.ShapeDtypeStruct((M, D), F32),
            jax.ShapeDtypeStruct((M, H * Dr), F32),
            jax.ShapeDtypeStruct((M, Dr), F32),
            jax.ShapeDtypeStruct((2, M, DCL), F32),
            jax.ShapeDtypeStruct((2, DCL, D), F32),
            jax.ShapeDtypeStruct((2, DCL, D), F32),
        ],
        scratch_shapes=[
            pltpu.VMEM((M, DCL), F32),
            pltpu.SemaphoreType.DMA((3,)),
            pltpu.SemaphoreType.DMA((3,)),
            pltpu.SemaphoreType.DMA((3,)),
        ],
        compiler_params=pltpu.CompilerParams(
            collective_id=0, vmem_limit_bytes=100 * 1024 * 1024),
    )(x2d, wdkv, wuk, wuv, wq, wqr, wkr)


HB = 4


def _attn_out_body(q_ref, qr_ref, kr_ref, c_ref, wuk_ref, wuv_ref, wo_ref,
                   o_ref, oj_scr):
    j = pl.program_id(0)
    dn = (((1,), (1,)), ((), ()))
    c0 = c_ref[0]
    c1 = c_ref[1]
    kh = (jnp.dot(c0, wuk_ref[0], preferred_element_type=F32)
          + jnp.dot(c1, wuk_ref[1], preferred_element_type=F32))
    vh = (jnp.dot(c0, wuv_ref[0], preferred_element_type=F32)
          + jnp.dot(c1, wuv_ref[1], preferred_element_type=F32))
    for b in range(B):
        rows = slice(b * S, (b + 1) * S)
        kr = kr_ref[rows, :]
        for t in range(HB):
            cols = slice(t * Dh, (t + 1) * Dh)
            q = q_ref[rows, cols]
            k = kh[rows, cols]
            v = vh[rows, cols]
            qr = qr_ref[rows, t * Dr:(t + 1) * Dr]
            s = lax.dot_general(q, k, dn, preferred_element_type=F32)
            s += lax.dot_general(qr, kr, dn, preferred_element_type=F32)
            p = jnp.exp(s)
            r = 1.0 / jnp.sum(p, axis=1, keepdims=True)
            oj_scr[rows, cols] = jnp.dot(
                p, v, preferred_element_type=F32) * r

    acc = jnp.dot(oj_scr[:, :], wo_ref[:, :], preferred_element_type=F32)

    @pl.when(j == 0)
    def _():
        o_ref[:, :] = acc

    @pl.when(j > 0)
    def _():
        o_ref[:, :] = o_ref[:, :] + acc


def _attention_out(q2d, qr2d, kr2d, c_st, wuk_st, wuv_st, wo):
    return pl.pallas_call(
        _attn_out_body,
        grid=(H // HB,),
        in_specs=[
            pl.BlockSpec((M, HB * Dh), lambda j: (0, j)),
            pl.BlockSpec((M, HB * Dr), lambda j: (0, j)),
            pl.BlockSpec((M, Dr), lambda j: (0, 0)),
            pl.BlockSpec((2, M, DCL), lambda j: (0, 0, 0)),
            pl.BlockSpec((2, DCL, HB * Dh), lambda j: (0, 0, j)),
            pl.BlockSpec((2, DCL, HB * Dh), lambda j: (0, 0, j)),
            pl.BlockSpec((HB * Dh, D), lambda j: (j, 0)),
        ],
        out_specs=pl.BlockSpec((M, D), lambda j: (0, 0)),
        out_shape=jax.ShapeDtypeStruct((M, D), F32),
        scratch_shapes=[pltpu.VMEM((M, HB * Dh), F32)],
        compiler_params=pltpu.CompilerParams(
            vmem_limit_bytes=60 * 1024 * 1024),
    )(q2d, qr2d, kr2d, c_st, wuk_st, wuv_st, wo)


def kernel(x, Wdkv, Wuk, Wuv, Wq, Wqr, Wkr, Wo):
    x2d = x.reshape(M, D)
    q2d, qr2d, kr2d, c_st, wuk_st, wuv_st = _fused_q_exchange(
        x2d, Wdkv, Wuk, Wuv, Wq, Wqr, Wkr)
    out2d = _attention_out(q2d, qr2d, kr2d, c_st, wuk_st, wuv_st, Wo)
    return out2d.reshape(B, S, D)


# device time: 195122 ns/iter; 1.5285x vs baseline; 1.0122x over previous
import jax
import jax.numpy as jnp
from jax import lax
from jax.experimental import pallas as pl
from jax.experimental.pallas import tpu as pltpu

B, S, H, Dh, Dr = 4, 256, 32, 128, 64
D = 4096
DCL = 128
M = B * S
F32 = jnp.float32
BF16 = jnp.bfloat16
import math
SCALE = float((Dh + Dr) ** -0.5 * math.log2(math.e))


def _matmul_body(x_ref, w_ref, o_ref):
    o_ref[:, :] = jnp.dot(x_ref[:, :], w_ref[:, :],
                          preferred_element_type=F32)


def _matmul(x, w, block_n):
    m, k = x.shape
    _, n = w.shape
    return pl.pallas_call(
        _matmul_body,
        grid=(n // block_n,),
        in_specs=[
            pl.BlockSpec((m, k), lambda j: (0, 0)),
            pl.BlockSpec((k, block_n), lambda j: (0, j)),
        ],
        out_specs=pl.BlockSpec((m, block_n), lambda j: (0, j)),
        out_shape=jax.ShapeDtypeStruct((m, n), F32),
        compiler_params=pltpu.CompilerParams(
            vmem_limit_bytes=48 * 1024 * 1024),
    )(x, w)


_QB = 256
_NJ_Q = D // _QB
_NJ = _NJ_Q + H * Dr // _QB


def _fused_body(x_ref, wdkv_ref, wuk_ref, wuv_ref, wq_ref, wqr_ref, wkr_ref,
                q_ref, qr_ref, kr_ref, c_out, wuk_out, wuv_out,
                c_scr, local_sems, send_sems, recv_sems):
    j = pl.program_id(0)
    my_x = lax.axis_index("x")
    my_y = lax.axis_index("y")
    my_z = lax.axis_index("z")
    peer = (my_x, my_y, 1 - my_z)

    def _descriptors():
        out = []
        for i, (src, dst) in enumerate(
            [(c_scr, c_out), (wuk_ref, wuk_out), (wuv_ref, wuv_out)]
        ):
            local = pltpu.make_async_copy(src, dst.at[my_z],
                                          local_sems.at[i])
            rdma = pltpu.make_async_remote_copy(
                src_ref=src,
                dst_ref=dst.at[my_z],
                send_sem=send_sems.at[i],
                recv_sem=recv_sems.at[i],
                device_id=peer,
                device_id_type=pl.DeviceIdType.MESH,
            )
            out.append((local, rdma))
        return out

    @pl.when(j == 0)
    def _():
        barrier = pltpu.get_barrier_semaphore()
        pl.semaphore_signal(barrier, inc=1, device_id=peer,
                            device_id_type=pl.DeviceIdType.MESH)
        pl.semaphore_wait(barrier, 1)

        c_scr[:, :] = jnp.dot(x_ref[:, :], wdkv_ref[:, :],
                              preferred_element_type=F32)
        for local, rdma in _descriptors():
            local.start()
            rdma.start()

    @pl.when(j < _NJ_Q)
    def _():
        q_ref[:, :] = (jnp.dot(x_ref[:, :], wq_ref[:, :],
                               preferred_element_type=F32)
                       * SCALE).astype(BF16)

    @pl.when(j >= _NJ_Q)
    def _():
        qr_ref[:, :] = (jnp.dot(x_ref[:, :], wqr_ref[:, :],
                                preferred_element_type=F32)
                        * SCALE).astype(BF16)

    @pl.when(j == _NJ_Q)
    def _():
        kr_ref[:, :] = jnp.dot(x_ref[:, :], wkr_ref[:, :],
                               preferred_element_type=F32).astype(BF16)

    @pl.when(j == _NJ - 1)
    def _():
        for local, rdma in _descriptors():
            local.wait()
            rdma.wait()


def _fused_q_exchange(x2d, wdkv, wuk, wuv, wq, wqr, wkr):
    return pl.pallas_call(
        _fused_body,
        grid=(_NJ,),
        in_specs=[
            pl.BlockSpec((M, D), lambda j: (0, 0)),
            pl.BlockSpec((D, DCL), lambda j: (0, 0)),
            pl.BlockSpec((DCL, D), lambda j: (0, 0)),
            pl.BlockSpec((DCL, D), lambda j: (0, 0)),
            pl.BlockSpec((D, _QB),
                         lambda j: (0, jnp.minimum(j, _NJ_Q - 1))),
            pl.BlockSpec((D, _QB),
                         lambda j: (0, jnp.maximum(j - _NJ_Q, 0))),
            pl.BlockSpec((D, Dr), lambda j: (0, 0)),
        ],
        out_specs=[
            pl.BlockSpec((M, _QB),
                         lambda j: (0, jnp.minimum(j, _NJ_Q - 1))),
            pl.BlockSpec((M, _QB),
                         lambda j: (0, jnp.maximum(j - _NJ_Q, 0))),
            pl.BlockSpec((M, Dr), lambda j: (0, 0)),
            pl.BlockSpec((2, M, DCL), lambda j: (0, 0, 0)),
            pl.BlockSpec((2, DCL, D), lambda j: (0, 0, 0)),
            pl.BlockSpec((2, DCL, D), lambda j: (0, 0, 0)),
        ],
        out_shape=[
            jax.ShapeDtypeStruct((M, D), BF16),
            jax.ShapeDtypeStruct((M, H * Dr), BF16),
            jax.ShapeDtypeStruct((M, Dr), BF16),
            jax.ShapeDtypeStruct((2, M, DCL), F32),
            jax.ShapeDtypeStruct((2, DCL, D), F32),
            jax.ShapeDtypeStruct((2, DCL, D), F32),
        ],
        scratch_shapes=[
            pltpu.VMEM((M, DCL), F32),
            pltpu.SemaphoreType.DMA((3,)),
            pltpu.SemaphoreType.DMA((3,)),
            pltpu.SemaphoreType.DMA((3,)),
        ],
        compiler_params=pltpu.CompilerParams(
            collective_id=0, vmem_limit_bytes=100 * 1024 * 1024),
    )(x2d, wdkv, wuk, wuv, wq, wqr, wkr)


HB = 4


def _attn_out_body(q_ref, qr_ref, kr_ref, c_ref, wuk_ref, wuv_ref, wo_ref,
                   o_ref, oj_scr):
    j = pl.program_id(0)
    dn = (((1,), (1,)), ((), ()))
    c0 = c_ref[0]
    c1 = c_ref[1]
    kh = (jnp.dot(c0, wuk_ref[0], preferred_element_type=F32)
          + jnp.dot(c1, wuk_ref[1], preferred_element_type=F32)
          ).astype(BF16)
    vh = (jnp.dot(c0, wuv_ref[0], preferred_element_type=F32)
          + jnp.dot(c1, wuv_ref[1], preferred_element_type=F32))
    for b in range(B):
        rows = slice(b * S, (b + 1) * S)
        kr = kr_ref[rows, :]
        for t in range(HB):
            cols = slice(t * Dh, (t + 1) * Dh)
            q = q_ref[rows, cols]
            k = kh[rows, cols]
            v = vh[rows, cols]
            qr = qr_ref[rows, t * Dr:(t + 1) * Dr]
            s = lax.dot_general(q, k, dn, preferred_element_type=F32)
            s += lax.dot_general(qr, kr, dn, preferred_element_type=F32)
            p = jnp.exp2(s)
            r = 1.0 / jnp.sum(p, axis=1, keepdims=True)
            oj_scr[rows, cols] = jnp.dot(
                p, v, preferred_element_type=F32) * r

    acc = jnp.dot(oj_scr[:, :], wo_ref[:, :], preferred_element_type=F32)

    @pl.when(j == 0)
    def _():
        o_ref[:, :] = acc

    @pl.when(j > 0)
    def _():
        o_ref[:, :] = o_ref[:, :] + acc


def _attention_out(q2d, qr2d, kr2d, c_st, wuk_st, wuv_st, wo):
    return pl.pallas_call(
        _attn_out_body,
        grid=(H // HB,),
        in_specs=[
            pl.BlockSpec((M, HB * Dh), lambda j: (0, j)),
            pl.BlockSpec((M, HB * Dr), lambda j: (0, j)),
            pl.BlockSpec((M, Dr), lambda j: (0, 0)),
            pl.BlockSpec((2, M, DCL), lambda j: (0, 0, 0)),
            pl.BlockSpec((2, DCL, HB * Dh), lambda j: (0, 0, j)),
            pl.BlockSpec((2, DCL, HB * Dh), lambda j: (0, 0, j)),
            pl.BlockSpec((HB * Dh, D), lambda j: (j, 0)),
        ],
        out_specs=pl.BlockSpec((M, D), lambda j: (0, 0)),
        out_shape=jax.ShapeDtypeStruct((M, D), F32),
        scratch_shapes=[pltpu.VMEM((M, HB * Dh), F32)],
        compiler_params=pltpu.CompilerParams(
            vmem_limit_bytes=60 * 1024 * 1024),
    )(q2d, qr2d, kr2d, c_st, wuk_st, wuv_st, wo)


def kernel(x, Wdkv, Wuk, Wuv, Wq, Wqr, Wkr, Wo):
    x2d = x.reshape(M, D)
    q2d, qr2d, kr2d, c_st, wuk_st, wuv_st = _fused_q_exchange(
        x2d, Wdkv, Wuk, Wuv, Wq, Wqr, Wkr)
    out2d = _attention_out(q2d, qr2d, kr2d, c_st, wuk_st, wuv_st, Wo)
    return out2d.reshape(B, S, D)
